# Initial kernel scaffold; baseline (speedup 1.0000x reference)
import jax
import jax.numpy as jnp
from jax.experimental import pallas as pl
from jax.experimental.pallas import tpu as pltpu

N = 16384
D = 256
H = 4
DH = 64
S = 8
B = 512
NB = N // B
KM_ITERS = 10

_INTERPRET = False


def _dot(a, b, dims):
    return jax.lax.dot_general(a, b, (dims, ((), ())),
                               preferred_element_type=jnp.float32)


def _kmeans4_body(x_ref, oh_ref, gates_ref, c_ref, sums_ref, cnts_ref):
    it = pl.program_id(0)
    b = pl.program_id(1)
    x = x_ref[...]

    @pl.when((it == 0) & (b == 0))
    def _():
        c_ref[0:4, :] = x[0:4, :]

    @pl.when(b == 0)
    def _():
        sums_ref[...] = jnp.zeros_like(sums_ref)
        cnts_ref[...] = jnp.zeros_like(cnts_ref)

    c4 = c_ref[0:4, :]
    xc = _dot(x, c4, ((1,), (1,)))
    c2 = _dot(jnp.ones((1, D), jnp.float32), c4 * c4, ((1,), (1,)))
    dist = c2 - 2.0 * xc
    best = dist[:, 0:1]
    bi = jnp.zeros((B, 1), jnp.float32)
    for k in range(1, 4):
        dk = dist[:, k:k + 1]
        upd = dk < best
        bi = jnp.where(upd, float(k), bi)
        best = jnp.where(upd, dk, best)
    lane8 = jax.lax.broadcasted_iota(jnp.float32, (B, S), 1)
    oh8 = (bi == lane8).astype(jnp.float32)
    oh_ref[...] = oh8
    sums_ref[...] = sums_ref[...] + _dot(oh8, x, ((0,), (0,)))
    cnts_ref[:, 0:1] = cnts_ref[:, 0:1] + _dot(
        oh8, jnp.ones((B, 1), jnp.float32), ((0,), (0,)))

    @pl.when(b == NB - 1)
    def _():
        c_ref[...] = sums_ref[...] / jnp.maximum(cnts_ref[:, 0:1], 1.0)

    @pl.when((it == KM_ITERS - 1) & (b == NB - 1))
    def _():
        cnt4 = cnts_ref[0:4, 0:1]
        gate4 = (cnt4 > 20.0).astype(jnp.float32)
        r8 = jax.lax.broadcasted_iota(jnp.int32, (S, 4), 0) // 2
        c4i = jax.lax.broadcasted_iota(jnp.int32, (S, 4), 1)
        expand = (r8 == c4i).astype(jnp.float32)
        g8 = _dot(expand, gate4, ((1,), (0,)))
        gates_ref[...] = jnp.broadcast_to(g8, (S, 128))


def _kmeans4(full):
    return pl.pallas_call(
        _kmeans4_body,
        grid=(KM_ITERS, NB),
        in_specs=[pl.BlockSpec((B, D), lambda it, b: (b, 0))],
        out_specs=[pl.BlockSpec((B, S), lambda it, b: (b, 0)),
                   pl.BlockSpec((S, 128), lambda it, b: (0, 0))],
        out_shape=[jax.ShapeDtypeStruct((N, S), jnp.float32),
                   jax.ShapeDtypeStruct((S, 128), jnp.float32)],
        scratch_shapes=[pltpu.VMEM((S, D), jnp.float32),
                        pltpu.VMEM((S, D), jnp.float32),
                        pltpu.VMEM((S, 128), jnp.float32)],
        interpret=_INTERPRET,
    )(full)


def _segkv_body(s_act, x_ref, oh_ref, wk_ref, wv_ref, kv_ref, ks_ref,
                akv_ref, ak_ref):
    b = pl.program_id(0)

    @pl.when(b == 0)
    def _():
        akv_ref[...] = jnp.zeros_like(akv_ref)
        ak_ref[...] = jnp.zeros_like(ak_ref)

    x = x_ref[...]
    k = _dot(x, wk_ref[...], ((1,), (1,)))
    kk = jnp.where(k > 0, k + 1.0, jnp.exp(k))
    v = _dot(x, wv_ref[...], ((1,), (1,)))
    oh = oh_ref[...]
    for s in range(s_act):
        ksg = kk * oh[:, s:s + 1]
        akv_ref[s * D:(s + 1) * D, :] = akv_ref[s * D:(s + 1) * D, :] + _dot(
            ksg, v, ((0,), (0,)))
        ak_ref[s:s + 1, :] = ak_ref[s:s + 1, :] + jnp.sum(
            ksg, axis=0, keepdims=True)

    @pl.when(b == NB - 1)
    def _():
        row = (jax.lax.broadcasted_iota(jnp.int32, (S * D, D), 0) % D) // DH
        col = jax.lax.broadcasted_iota(jnp.int32, (S * D, D), 1) // DH
        bd = (row == col).astype(jnp.float32)
        kv_ref[...] = akv_ref[...] * bd
        ks_ref[...] = ak_ref[...]


def _segkv(x, oh, wk, wv, s_act):
    body = lambda *refs: _segkv_body(s_act, *refs)
    return pl.pallas_call(
        body,
        grid=(NB,),
        in_specs=[pl.BlockSpec((B, D), lambda b: (b, 0)),
                  pl.BlockSpec((B, S), lambda b: (b, 0)),
                  pl.BlockSpec((D, D), lambda b: (0, 0)),
                  pl.BlockSpec((D, D), lambda b: (0, 0))],
        out_specs=[pl.BlockSpec((S * D, D), lambda b: (0, 0)),
                   pl.BlockSpec((S, D), lambda b: (0, 0))],
        out_shape=[jax.ShapeDtypeStruct((S * D, D), jnp.float32),
                   jax.ShapeDtypeStruct((S, D), jnp.float32)],
        scratch_shapes=[pltpu.VMEM((S * D, D), jnp.float32),
                        pltpu.VMEM((S, D), jnp.float32)],
        interpret=_INTERPRET,
    )(x, oh, wk, wv)


def _apply_body(s_act, final, *refs):
    if final:
        (x_ref, oh_ref, kv_ref, ks_ref, wq_ref, wm_ref, m1_ref, m2_ref,
         ln_ref, orig_ref, gates_ref, y_ref) = refs
    else:
        (x_ref, oh_ref, kv_ref, ks_ref, wq_ref, wm_ref, m1_ref, m2_ref,
         ln_ref, y_ref) = refs
    x = x_ref[...]
    oh = oh_ref[...]
    q = _dot(x, wq_ref[...], ((1,), (1,)))
    qq = jnp.where(q > 0, q + 1.0, jnp.exp(q))
    o = jnp.zeros((B, D), jnp.float32)
    for s in range(s_act):
        o = o + oh[:, s:s + 1] * _dot(qq, kv_ref[s * D:(s + 1) * D, :],
                                      ((1,), (0,)))
    ksel = _dot(oh, ks_ref[...], ((1,), (0,)))
    ehr = (jax.lax.broadcasted_iota(jnp.int32, (D, H), 0) // DH ==
           jax.lax.broadcasted_iota(jnp.int32, (D, H), 1)).astype(jnp.float32)
    den = _dot(qq * ksel, ehr, ((1,), (0,))) + 1e-6
    zf = _dot(1.0 / den, ehr, ((1,), (1,)))
    msg = _dot(o * zf, wm_ref[...], ((1,), (1,)))
    mu = jnp.mean(msg, axis=1, keepdims=True)
    var = jnp.mean((msg - mu) ** 2, axis=1, keepdims=True)
    msg = (msg - mu) / jnp.sqrt(var + 1e-5) * ln_ref[0:1, :] + ln_ref[1:2, :]
    h = _dot(x, m1_ref[:, 0:D], ((1,), (1,))) + _dot(
        msg, m1_ref[:, D:2 * D], ((1,), (1,)))
    h = jnp.maximum(h, 0.0)
    h2 = _dot(h, m2_ref[...], ((1,), (1,)))
    mu2 = jnp.mean(h2, axis=1, keepdims=True)
    var2 = jnp.mean((h2 - mu2) ** 2, axis=1, keepdims=True)
    msg2 = (h2 - mu2) / jnp.sqrt(var2 + 1e-5) * ln_ref[2:3, :] + ln_ref[3:4, :]
    y = x + msg2
    if final:
        g = _dot(oh, gates_ref[:, 0:1], ((1,), (0,)))
        y = g * y + (1.0 - g) * orig_ref[...]
    y_ref[...] = y


def _apply(x, oh, kv, ks, wq, wm, m1, m2, ln, s_act, orig=None, gates=None):
    final = orig is not None
    body = lambda *refs: _apply_body(s_act, final, *refs)
    in_specs = [pl.BlockSpec((B, D), lambda b: (b, 0)),
                pl.BlockSpec((B, S), lambda b: (b, 0)),
                pl.BlockSpec((S * D, D), lambda b: (0, 0)),
                pl.BlockSpec((S, D), lambda b: (0, 0)),
                pl.BlockSpec((D, D), lambda b: (0, 0)),
                pl.BlockSpec((D, D), lambda b: (0, 0)),
                pl.BlockSpec((2 * D, 2 * D), lambda b: (0, 0)),
                pl.BlockSpec((D, 2 * D), lambda b: (0, 0)),
                pl.BlockSpec((8, D), lambda b: (0, 0))]
    args = [x, oh, kv, ks, wq, wm, m1, m2, ln]
    if final:
        in_specs += [pl.BlockSpec((B, D), lambda b: (b, 0)),
                     pl.BlockSpec((S, 128), lambda b: (0, 0))]
        args += [orig, gates]
    return pl.pallas_call(
        body,
        grid=(NB,),
        in_specs=in_specs,
        out_specs=pl.BlockSpec((B, D), lambda b: (b, 0)),
        out_shape=jax.ShapeDtypeStruct((N, D), jnp.float32),
        interpret=_INTERPRET,
    )(*args)


def _subkm_body(x_ref, oh4_ref, oh8_ref, c_ref, sums_ref, cnts_ref, seen_ref):
    it = pl.program_id(0)
    b = pl.program_id(1)
    x = x_ref[...]
    oh4 = oh4_ref[...][:, 0:4]
    r4 = jax.lax.broadcasted_iota(jnp.int32, (4, S), 0)
    c8 = jax.lax.broadcasted_iota(jnp.int32, (4, S), 1)
    e2 = (c8 // 2 == r4).astype(jnp.float32)
    oh48 = _dot(oh4, e2, ((1,), (0,)))
    lane8 = jax.lax.broadcasted_iota(jnp.int32, (B, S), 1)

    @pl.when(b == 0)
    def _():
        sums_ref[...] = jnp.zeros_like(sums_ref)
        cnts_ref[...] = jnp.zeros_like(cnts_ref)

    @pl.when(it == 0)
    def _():
        @pl.when(b == 0)
        def _():
            seen_ref[...] = jnp.zeros_like(seen_ref)
        ri = jax.lax.broadcasted_iota(jnp.int32, (B, B), 0)
        ci = jax.lax.broadcasted_iota(jnp.int32, (B, B), 1)
        lower = (ri >= ci).astype(jnp.float32)
        cum = _dot(lower, oh4, ((1,), (0,)))
        grank = cum + seen_ref[0:1, 0:4]
        grank8 = _dot(grank, e2, ((1,), (0,)))
        tgt = (lane8 % 2 + 1).astype(jnp.float32)
        w8 = oh48 * (grank8 == tgt).astype(jnp.float32)
        sums_ref[...] = sums_ref[...] + _dot(w8, x, ((0,), (0,)))
        seen_ref[0:1, 0:4] = seen_ref[0:1, 0:4] + jnp.sum(
            oh4, axis=0, keepdims=True)
        oh8_ref[...] = w8

        @pl.when(b == NB - 1)
        def _():
            c_ref[...] = sums_ref[...]

    @pl.when(it > 0)
    def _():
        cc = c_ref[...]
        xc = _dot(x, cc, ((1,), (1,)))
        c2 = _dot(jnp.ones((1, D), jnp.float32), cc * cc, ((1,), (1,)))
        dist = c2 - 2.0 * xc
        sr = jax.lax.broadcasted_iota(jnp.int32, (S, S), 0)
        sc = jax.lax.broadcasted_iota(jnp.int32, (S, S), 1)
        swap = ((sr ^ 1) == sc).astype(jnp.float32)
        dsw = _dot(dist, swap, ((1,), (0,)))
        even = (lane8 % 2) == 0
        sel = jnp.where(even, dist <= dsw, dist < dsw)
        oh8 = oh48 * sel.astype(jnp.float32)
        oh8_ref[...] = oh8
        sums_ref[...] = sums_ref[...] + _dot(oh8, x, ((0,), (0,)))
        cnts_ref[:, 0:1] = cnts_ref[:, 0:1] + _dot(
            oh8, jnp.ones((B, 1), jnp.float32), ((0,), (0,)))

        @pl.when(b == NB - 1)
        def _():
            c_ref[...] = sums_ref[...] / jnp.maximum(cnts_ref[:, 0:1], 1.0)


def _subkm(d0, oh4):
    return pl.pallas_call(
        _subkm_body,
        grid=(KM_ITERS + 1, NB),
        in_specs=[pl.BlockSpec((B, D), lambda it, b: (b, 0)),
                  pl.BlockSpec((B, S), lambda it, b: (b, 0))],
        out_specs=pl.BlockSpec((B, S), lambda it, b: (b, 0)),
        out_shape=jax.ShapeDtypeStruct((N, S), jnp.float32),
        scratch_shapes=[pltpu.VMEM((S, D), jnp.float32),
                        pltpu.VMEM((S, D), jnp.float32),
                        pltpu.VMEM((S, 128), jnp.float32),
                        pltpu.VMEM((S, 128), jnp.float32)],
        interpret=_INTERPRET,
    )(d0, oh4)


def kernel(desc2d, desc3d,
           l1_q, l1_k, l1_v, l1_merge, l1_mlp1, l1_mlp2,
           l1_n1g, l1_n1b, l1_n2g, l1_n2b,
           l2_q, l2_k, l2_v, l2_merge, l2_mlp1, l2_mlp2,
           l2_n1g, l2_n1b, l2_n2g, l2_n2b):
    n2 = desc2d.shape[0]
    full = jnp.concatenate([desc2d, desc3d], axis=0)
    ln1 = jnp.concatenate([l1_n1g.reshape(1, D), l1_n1b.reshape(1, D),
                           l1_n2g.reshape(1, D), l1_n2b.reshape(1, D),
                           jnp.zeros((4, D), jnp.float32)], axis=0)
    ln2 = jnp.concatenate([l2_n1g.reshape(1, D), l2_n1b.reshape(1, D),
                           l2_n2g.reshape(1, D), l2_n2b.reshape(1, D),
                           jnp.zeros((4, D), jnp.float32)], axis=0)
    oh4, gates = _kmeans4(full)
    kv1, ks1 = _segkv(full, oh4, l1_k, l1_v, s_act=4)
    d0 = _apply(full, oh4, kv1, ks1, l1_q, l1_merge, l1_mlp1, l1_mlp2, ln1,
                s_act=4)
    oh8 = _subkm(d0, oh4)
    kv2, ks2 = _segkv(d0, oh8, l2_k, l2_v, s_act=8)
    out = _apply(d0, oh8, kv2, ks2, l2_q, l2_merge, l2_mlp1, l2_mlp2, ln2,
                 s_act=8, orig=full, gates=gates)
    return out[:n2, :], out[n2:, :]


# speedup vs baseline: 5.2626x; 5.2626x over previous
import jax
import jax.numpy as jnp
from jax.experimental import pallas as pl
from jax.experimental.pallas import tpu as pltpu

N = 16384
D = 256
H = 4
DH = 64
S = 8
B = 512
NB = N // B
KM_ITERS = 10
CH = 2048
NCH = N // CH

_INTERPRET = False


def _dotbf(a, b, dims):
    return jax.lax.dot_general(a.astype(jnp.bfloat16), b.astype(jnp.bfloat16),
                               (dims, ((), ())),
                               preferred_element_type=jnp.float32)


def _dothi(a, b, dims):
    return jax.lax.dot_general(a, b, (dims, ((), ())),
                               precision=jax.lax.Precision.HIGHEST,
                               preferred_element_type=jnp.float32)


def _elu1(x):
    return jnp.where(x > 0, x, jnp.exp(x) - 1.0) + 1.0


def _kmeans4_body(x_ref, oh_ref, gates_ref, cnts_ref, c_ref):
    it = pl.program_id(0)

    @pl.when(it == 0)
    def _():
        c_ref[...] = jnp.zeros_like(c_ref)
        c_ref[0:4, :] = x_ref[0:4, :]

    cc = c_ref[...]
    c2 = _dothi(jnp.ones((1, D), jnp.float32), cc * cc, ((1,), (1,)))
    lane8 = jax.lax.broadcasted_iota(jnp.int32, (CH, S), 1)
    upper = (jax.lax.broadcasted_iota(jnp.int32, (S, S), 0) <=
             jax.lax.broadcasted_iota(jnp.int32, (S, S), 1)).astype(jnp.float32)

    def chunk(i, carry):
        sums, cnt = carry
        x = x_ref[pl.ds(i * CH, CH), :]
        x2 = jnp.sum(x * x, axis=1, keepdims=True)
        xc = _dotbf(x, cc, ((1,), (1,)))
        dist = x2 - 2.0 * xc + c2
        dist = jnp.where(lane8 < 4, dist, 3e38)
        best = jnp.min(dist, axis=1, keepdims=True)
        eq = (dist == best).astype(jnp.float32)
        cum = _dothi(eq, upper, ((1,), (0,)))
        oh8 = eq * (cum == 1.0).astype(jnp.float32)
        oh_ref[pl.ds(i * CH, CH), :] = oh8
        sums = sums + _dothi(oh8, x, ((0,), (0,)))
        cnt = cnt + _dothi(oh8, jnp.ones((CH, 1), jnp.float32), ((0,), (0,)))
        return sums, cnt

    sums, cnt = jax.lax.fori_loop(
        0, NCH, chunk,
        (jnp.zeros((S, D), jnp.float32), jnp.zeros((S, 1), jnp.float32)))
    c_ref[...] = sums / jnp.maximum(cnt, 1.0)

    @pl.when(it == KM_ITERS - 1)
    def _():
        cnts_ref[...] = jnp.broadcast_to(cnt, (S, 128))
        gate4 = (cnt[0:4, :] > 20.0).astype(jnp.float32)
        r8 = jax.lax.broadcasted_iota(jnp.int32, (S, 4), 0) // 2
        c4i = jax.lax.broadcasted_iota(jnp.int32, (S, 4), 1)
        expand = (r8 == c4i).astype(jnp.float32)
        g8 = _dothi(expand, gate4, ((1,), (0,)))
        gates_ref[...] = jnp.broadcast_to(g8, (S, 128))


def _kmeans4(full):
    return pl.pallas_call(
        _kmeans4_body,
        grid=(KM_ITERS,),
        in_specs=[pl.BlockSpec((N, D), lambda it: (0, 0))],
        out_specs=[pl.BlockSpec((N, S), lambda it: (0, 0)),
                   pl.BlockSpec((S, 128), lambda it: (0, 0)),
                   pl.BlockSpec((S, 128), lambda it: (0, 0))],
        out_shape=[jax.ShapeDtypeStruct((N, S), jnp.float32),
                   jax.ShapeDtypeStruct((S, 128), jnp.float32),
                   jax.ShapeDtypeStruct((S, 128), jnp.float32)],
        scratch_shapes=[pltpu.VMEM((S, D), jnp.float32)],
        interpret=_INTERPRET,
    )(full)


def _segkv_body(s_act, x_ref, oh_ref, cnts_ref, wk_ref, wv_ref, kv_ref,
                ks_ref, akv_ref, ak_ref):
    b = pl.program_id(0)

    @pl.when(b == 0)
    def _():
        akv_ref[...] = jnp.zeros_like(akv_ref)
        ak_ref[...] = jnp.zeros_like(ak_ref)

    x = x_ref[...]
    kk = _elu1(_dotbf(x, wk_ref[...], ((1,), (1,))))
    v = _dotbf(x, wv_ref[...], ((1,), (1,)))
    oh = oh_ref[...]
    cntsel = _dothi(oh, cnts_ref[:, 0:1], ((1,), (0,)))
    vd = v / jnp.maximum(cntsel, 1.0)
    for s in range(s_act):
        ksg = kk * oh[:, s:s + 1]
        akv_ref[s * D:(s + 1) * D, :] = akv_ref[s * D:(s + 1) * D, :] + _dotbf(
            ksg, vd, ((0,), (0,)))
        ak_ref[s:s + 1, :] = ak_ref[s:s + 1, :] + jnp.sum(
            ksg, axis=0, keepdims=True)

    @pl.when(b == NB - 1)
    def _():
        row = (jax.lax.broadcasted_iota(jnp.int32, (S * D, D), 0) % D) // DH
        col = jax.lax.broadcasted_iota(jnp.int32, (S * D, D), 1) // DH
        bd = (row == col).astype(jnp.float32)
        kv_ref[...] = akv_ref[...] * bd
        ks_ref[...] = ak_ref[...]


def _segkv(x, oh, cnts, wk, wv, s_act):
    body = lambda *refs: _segkv_body(s_act, *refs)
    return pl.pallas_call(
        body,
        grid=(NB,),
        in_specs=[pl.BlockSpec((B, D), lambda b: (b, 0)),
                  pl.BlockSpec((B, S), lambda b: (b, 0)),
                  pl.BlockSpec((S, 128), lambda b: (0, 0)),
                  pl.BlockSpec((D, D), lambda b: (0, 0)),
                  pl.BlockSpec((D, D), lambda b: (0, 0))],
        out_specs=[pl.BlockSpec((S * D, D), lambda b: (0, 0)),
                   pl.BlockSpec((S, D), lambda b: (0, 0))],
        out_shape=[jax.ShapeDtypeStruct((S * D, D), jnp.float32),
                   jax.ShapeDtypeStruct((S, D), jnp.float32)],
        scratch_shapes=[pltpu.VMEM((S * D, D), jnp.float32),
                        pltpu.VMEM((S, D), jnp.float32)],
        interpret=_INTERPRET,
    )(x, oh, cnts, wk, wv)


def _apply_body(s_act, final, *refs):
    if final:
        (x_ref, oh_ref, kv_ref, ks_ref, cnts_ref, wq_ref, wm_ref, m1_ref,
         m2_ref, ln_ref, orig_ref, gates_ref, y_ref) = refs
    else:
        (x_ref, oh_ref, kv_ref, ks_ref, cnts_ref, wq_ref, wm_ref, m1_ref,
         m2_ref, ln_ref, y_ref) = refs
    x = x_ref[...]
    oh = oh_ref[...]
    qq = _elu1(_dotbf(x, wq_ref[...], ((1,), (1,))))
    o = jnp.zeros((B, D), jnp.float32)
    for s in range(s_act):
        o = o + oh[:, s:s + 1] * _dotbf(qq, kv_ref[s * D:(s + 1) * D, :],
                                        ((1,), (0,)))
    ksel = _dothi(oh, ks_ref[...], ((1,), (0,)))
    ehr = (jax.lax.broadcasted_iota(jnp.int32, (D, H), 0) // DH ==
           jax.lax.broadcasted_iota(jnp.int32, (D, H), 1)).astype(jnp.float32)
    p = (qq.astype(jnp.bfloat16).astype(jnp.float32) *
         ksel.astype(jnp.bfloat16).astype(jnp.float32))
    den = _dothi(p, ehr, ((1,), (0,))) + 1e-6
    zf = _dothi(1.0 / den, ehr, ((1,), (1,)))
    cntsel = _dothi(oh, cnts_ref[:, 0:1], ((1,), (0,)))
    msg0 = (o * zf) * cntsel
    msg = _dotbf(msg0, wm_ref[...], ((1,), (1,)))
    mu = jnp.mean(msg, axis=1, keepdims=True)
    var = jnp.mean((msg - mu) ** 2, axis=1, keepdims=True)
    msg = (msg - mu) / jnp.sqrt(var + 1e-5) * ln_ref[0:1, :] + ln_ref[1:2, :]
    h = _dotbf(x, m1_ref[:, 0:D], ((1,), (1,))) + _dotbf(
        msg, m1_ref[:, D:2 * D], ((1,), (1,)))
    h = jnp.maximum(h, 0.0)
    h2 = _dotbf(h, m2_ref[...], ((1,), (1,)))
    mu2 = jnp.mean(h2, axis=1, keepdims=True)
    var2 = jnp.mean((h2 - mu2) ** 2, axis=1, keepdims=True)
    msg2 = (h2 - mu2) / jnp.sqrt(var2 + 1e-5) * ln_ref[2:3, :] + ln_ref[3:4, :]
    y = x + msg2
    if final:
        g = _dothi(oh, gates_ref[:, 0:1], ((1,), (0,)))
        y = g * y + (1.0 - g) * orig_ref[...]
    y_ref[...] = y


def _apply(x, oh, kv, ks, cnts, wq, wm, m1, m2, ln, s_act,
           orig=None, gates=None):
    final = orig is not None
    body = lambda *refs: _apply_body(s_act, final, *refs)
    in_specs = [pl.BlockSpec((B, D), lambda b: (b, 0)),
                pl.BlockSpec((B, S), lambda b: (b, 0)),
                pl.BlockSpec((S * D, D), lambda b: (0, 0)),
                pl.BlockSpec((S, D), lambda b: (0, 0)),
                pl.BlockSpec((S, 128), lambda b: (0, 0)),
                pl.BlockSpec((D, D), lambda b: (0, 0)),
                pl.BlockSpec((D, D), lambda b: (0, 0)),
                pl.BlockSpec((2 * D, 2 * D), lambda b: (0, 0)),
                pl.BlockSpec((D, 2 * D), lambda b: (0, 0)),
                pl.BlockSpec((8, D), lambda b: (0, 0))]
    args = [x, oh, kv, ks, cnts, wq, wm, m1, m2, ln]
    if final:
        in_specs += [pl.BlockSpec((B, D), lambda b: (b, 0)),
                     pl.BlockSpec((S, 128), lambda b: (0, 0))]
        args += [orig, gates]
    return pl.pallas_call(
        body,
        grid=(NB,),
        in_specs=in_specs,
        out_specs=pl.BlockSpec((B, D), lambda b: (b, 0)),
        out_shape=jax.ShapeDtypeStruct((N, D), jnp.float32),
        interpret=_INTERPRET,
    )(*args)


def _subkm_body(x_ref, oh4_ref, oh8_ref, cnts_ref, c_ref):
    it = pl.program_id(0)
    r4 = jax.lax.broadcasted_iota(jnp.int32, (4, S), 0)
    c8 = jax.lax.broadcasted_iota(jnp.int32, (4, S), 1)
    e2 = (c8 // 2 == r4).astype(jnp.float32)

    @pl.when(it == 0)
    def _():
        def scan1(i, carry):
            m0, m1 = carry
            oh4 = oh4_ref[pl.ds(i * CH, CH), :][:, 0:4]
            ii = (jax.lax.broadcasted_iota(jnp.int32, (CH, 4), 0)
                  + i * CH).astype(jnp.float32)
            iim = jnp.where(oh4 > 0, ii, 1e9)
            cm0 = jnp.min(iim, axis=0, keepdims=True)
            cm1 = jnp.min(jnp.where(iim > cm0, iim, 1e9), axis=0,
                          keepdims=True)
            nm0 = jnp.minimum(m0, cm0)
            nm1 = jnp.minimum(jnp.minimum(jnp.maximum(m0, cm0), m1), cm1)
            return nm0, nm1

        big = jnp.full((1, 4), 1e9, jnp.float32)
        m0, m1 = jax.lax.fori_loop(0, NCH, scan1, (big, big))
        ee = (2 * r4 == c8).astype(jnp.float32)
        eo = (2 * r4 + 1 == c8).astype(jnp.float32)
        m8 = _dothi(m0, ee, ((1,), (0,))) + _dothi(m1, eo, ((1,), (0,)))

        def scan2(i, sums):
            oh4 = oh4_ref[pl.ds(i * CH, CH), :][:, 0:4]
            x = x_ref[pl.ds(i * CH, CH), :]
            oh48 = _dothi(oh4, e2, ((1,), (0,)))
            ii8 = (jax.lax.broadcasted_iota(jnp.int32, (CH, S), 0)
                   + i * CH).astype(jnp.float32)
            w8 = oh48 * (ii8 == m8).astype(jnp.float32)
            return sums + _dothi(w8, x, ((0,), (0,)))

        c_ref[...] = jax.lax.fori_loop(
            0, NCH, scan2, jnp.zeros((S, D), jnp.float32))

    @pl.when(it > 0)
    def _():
        cc = c_ref[...]
        c2 = _dothi(jnp.ones((1, D), jnp.float32), cc * cc, ((1,), (1,)))
        sr = jax.lax.broadcasted_iota(jnp.int32, (S, S), 0)
        sc = jax.lax.broadcasted_iota(jnp.int32, (S, S), 1)
        swap = ((sr ^ 1) == sc).astype(jnp.float32)
        lane8 = jax.lax.broadcasted_iota(jnp.int32, (CH, S), 1)
        evenf = ((lane8 % 2) == 0).astype(jnp.float32)

        def chunk(i, carry):
            sums, cnt = carry
            x = x_ref[pl.ds(i * CH, CH), :]
            oh4 = oh4_ref[pl.ds(i * CH, CH), :][:, 0:4]
            oh48 = _dothi(oh4, e2, ((1,), (0,)))
            x2 = jnp.sum(x * x, axis=1, keepdims=True)
            xc = _dotbf(x, cc, ((1,), (1,)))
            dist = x2 - 2.0 * xc + c2
            dsw = _dothi(dist, swap, ((1,), (0,)))
            sel = (evenf * (dist <= dsw).astype(jnp.float32) +
                   (1.0 - evenf) * (dist < dsw).astype(jnp.float32))
            oh8 = oh48 * sel
            oh8_ref[pl.ds(i * CH, CH), :] = oh8
            sums = sums + _dothi(oh8, x, ((0,), (0,)))
            cnt = cnt + _dothi(oh8, jnp.ones((CH, 1), jnp.float32),
                               ((0,), (0,)))
            return sums, cnt

        sums, cnt = jax.lax.fori_loop(
            0, NCH, chunk,
            (jnp.zeros((S, D), jnp.float32), jnp.zeros((S, 1), jnp.float32)))
        c_ref[...] = sums / jnp.maximum(cnt, 1.0)

        @pl.when(it == KM_ITERS)
        def _():
            cnts_ref[...] = jnp.broadcast_to(cnt, (S, 128))


def _subkm(d0, oh4):
    return pl.pallas_call(
        _subkm_body,
        grid=(KM_ITERS + 1,),
        in_specs=[pl.BlockSpec((N, D), lambda it: (0, 0)),
                  pl.BlockSpec((N, S), lambda it: (0, 0))],
        out_specs=[pl.BlockSpec((N, S), lambda it: (0, 0)),
                   pl.BlockSpec((S, 128), lambda it: (0, 0))],
        out_shape=[jax.ShapeDtypeStruct((N, S), jnp.float32),
                   jax.ShapeDtypeStruct((S, 128), jnp.float32)],
        scratch_shapes=[pltpu.VMEM((S, D), jnp.float32)],
        interpret=_INTERPRET,
    )(d0, oh4)


def kernel(desc2d, desc3d,
           l1_q, l1_k, l1_v, l1_merge, l1_mlp1, l1_mlp2,
           l1_n1g, l1_n1b, l1_n2g, l1_n2b,
           l2_q, l2_k, l2_v, l2_merge, l2_mlp1, l2_mlp2,
           l2_n1g, l2_n1b, l2_n2g, l2_n2b):
    n2 = desc2d.shape[0]
    full = jnp.concatenate([desc2d, desc3d], axis=0)
    ln1 = jnp.concatenate([l1_n1g.reshape(1, D), l1_n1b.reshape(1, D),
                           l1_n2g.reshape(1, D), l1_n2b.reshape(1, D),
                           jnp.zeros((4, D), jnp.float32)], axis=0)
    ln2 = jnp.concatenate([l2_n1g.reshape(1, D), l2_n1b.reshape(1, D),
                           l2_n2g.reshape(1, D), l2_n2b.reshape(1, D),
                           jnp.zeros((4, D), jnp.float32)], axis=0)
    oh4, gates, cnts4 = _kmeans4(full)
    kv1, ks1 = _segkv(full, oh4, cnts4, l1_k, l1_v, s_act=4)
    d0 = _apply(full, oh4, kv1, ks1, cnts4, l1_q, l1_merge, l1_mlp1, l1_mlp2,
                ln1, s_act=4)
    oh8, cnts8 = _subkm(d0, oh4)
    kv2, ks2 = _segkv(d0, oh8, cnts8, l2_k, l2_v, s_act=8)
    out = _apply(d0, oh8, kv2, ks2, cnts8, l2_q, l2_merge, l2_mlp1, l2_mlp2,
                 ln2, s_act=8, orig=full, gates=gates)
    return out[:n2, :], out[n2:, :]
